# baseline (speedup 1.0000x reference)
import functools

import jax
import jax.numpy as jnp
from jax import lax
from jax.experimental import pallas as pl
from jax.experimental.pallas import tpu as pltpu
from jax.experimental.pallas import tpu_sc as plsc

B, T, D, H, L, E, K, V, DFF = 8, 128, 128, 4, 4, 16, 8, 262144, 512
HD = D // H
N = B * T

_NC, _NS = 2, 16
_NW = _NC * _NS
_BPW = N // _NW


def _sc_gather(table, idx):
    mesh = plsc.VectorSubcoreMesh(
        core_axis_name="c", subcore_axis_name="s",
        num_cores=_NC, num_subcores=_NS)

    @functools.partial(
        pl.kernel,
        out_type=jax.ShapeDtypeStruct((N, D), jnp.float32),
        mesh=mesh,
        scratch_types=[
            pltpu.VMEM((_BPW,), jnp.int32),
            pltpu.VMEM((_BPW, D), jnp.float32),
            pltpu.SemaphoreType.DMA,
        ],
    )
    def gather_kernel(table_hbm, idx_hbm, out_hbm, idx_v, rows_v, sem):
        wid = lax.axis_index("s") * _NC + lax.axis_index("c")
        base = wid * _BPW
        pltpu.sync_copy(idx_hbm.at[pl.ds(base, _BPW)], idx_v)
        pltpu.async_copy(table_hbm.at[idx_v], rows_v, sem).wait()
        pltpu.sync_copy(rows_v, out_hbm.at[pl.ds(base, _BPW)])

    return gather_kernel(table, idx)


def _ln(x, g, b):
    m = jnp.mean(x, axis=-1, keepdims=True)
    c = x - m
    v = jnp.mean(c * c, axis=-1, keepdims=True)
    return c * lax.rsqrt(v + 1e-5) * g + b


def _tc_body(emb_ref, pos_ref, wqkv_ref, wo_ref, rtr_ref, w1_ref, w2_ref,
             ln0g_ref, ln0b_ref, lnag_ref, lnab_ref, lnbg_ref, lnbb_ref,
             lnfg_ref, lnfb_ref, out_ref, h_ref):
    l = pl.program_id(0)

    @pl.when(l == 0)
    def _():
        h_ref[...] = _ln(emb_ref[...] + pos_ref[...],
                         ln0g_ref[...], ln0b_ref[...])

    h = h_ref[...]
    x = _ln(h, lnag_ref[0], lnab_ref[0])
    wqkv = wqkv_ref[0]
    wo = wo_ref[0]
    row = lax.broadcasted_iota(jnp.int32, (T, T), 0)
    col = lax.broadcasted_iota(jnp.int32, (T, T), 1)
    causal = row >= col
    scale = 1.0 / (HD ** 0.5)
    for b in range(B):
        xb = x[b * T:(b + 1) * T]
        qkv = jnp.dot(xb, wqkv, preferred_element_type=jnp.float32)
        obs = []
        for hh in range(H):
            q = qkv[:, hh * HD:(hh + 1) * HD]
            k = qkv[:, D + hh * HD:D + (hh + 1) * HD]
            v = qkv[:, 2 * D + hh * HD:2 * D + (hh + 1) * HD]
            att = lax.dot_general(q, k, (((1,), (1,)), ((), ())),
                                  preferred_element_type=jnp.float32) * scale
            att = jnp.where(causal, att, jnp.float32(-1e9))
            att = jax.nn.softmax(att, axis=-1)
            obs.append(jnp.dot(att, v, preferred_element_type=jnp.float32))
        ob = jnp.concatenate(obs, axis=-1)
        h_ref[b * T:(b + 1) * T, :] = (
            h[b * T:(b + 1) * T]
            + jnp.dot(ob, wo, preferred_element_type=jnp.float32))

    h = h_ref[...]
    x = _ln(h, lnbg_ref[0], lnbb_ref[0])
    logits = jnp.dot(x, rtr_ref[0], preferred_element_type=jnp.float32)
    probs = jax.nn.softmax(logits, axis=-1)
    masked = probs
    sel = jnp.zeros_like(probs)
    pos16 = lax.broadcasted_iota(jnp.int32, (N, E), 1)
    for _ in range(K):
        m = jnp.max(masked, axis=-1, keepdims=True)
        cand = masked == m
        candpos = jnp.where(cand, pos16, E)
        first = candpos == jnp.min(candpos, axis=-1, keepdims=True)
        sel = jnp.where(first, probs, sel)
        masked = jnp.where(first, jnp.float32(-1.0), masked)
    gates = sel / (jnp.sum(sel, axis=-1, keepdims=True) + 1e-9)

    acc = h
    xb = x.astype(jnp.bfloat16)
    for e in range(E):
        mid = jax.nn.gelu(jnp.dot(xb, w1_ref[0, e],
                                  preferred_element_type=jnp.float32))
        gm = (mid * gates[:, e:e + 1]).astype(jnp.bfloat16)
        acc = acc + jnp.dot(gm, w2_ref[0, e],
                            preferred_element_type=jnp.float32)
    h_ref[...] = acc

    @pl.when(l == L - 1)
    def _():
        out_ref[...] = _ln(acc, lnfg_ref[...], lnfb_ref[...])


def _tc_forward(emb, pos_t, Wqkv, Wo, router, W1, W2,
                ln0g, ln0b, lnag, lnab, lnbg, lnbb, lnfg, lnfb,
                interpret=False):
    full2d = pl.BlockSpec((N, D), lambda l: (0, 0))
    perl = lambda shape: pl.BlockSpec(shape, lambda l: (l,) + (0,) * (len(shape) - 1))
    row0 = pl.BlockSpec((1, D), lambda l: (0, 0))
    rowl = pl.BlockSpec((1, 1, D), lambda l: (l, 0, 0))
    return pl.pallas_call(
        _tc_body,
        grid=(L,),
        in_specs=[
            full2d, full2d,
            perl((1, D, 3 * D)),
            perl((1, D, D)),
            perl((1, D, E)),
            perl((1, E, D, DFF)),
            perl((1, E, DFF, D)),
            row0, row0, rowl, rowl, rowl, rowl, row0, row0,
        ],
        out_specs=full2d,
        out_shape=jax.ShapeDtypeStruct((N, D), jnp.float32),
        scratch_shapes=[pltpu.VMEM((N, D), jnp.float32)],
        interpret=interpret,
    )(emb, pos_t, Wqkv, Wo, router, W1, W2,
      ln0g, ln0b, lnag, lnab, lnbg, lnbb, lnfg, lnfb)


def kernel(input_ids, tok_emb, pos_emb, Wqkv, Wo, router, W1, W2, ln_g, ln_b):
    ids = input_ids.reshape(N).astype(jnp.int32)
    emb = _sc_gather(tok_emb, ids)
    pos_t = jnp.broadcast_to(pos_emb[None], (B, T, D)).reshape(N, D)
    out = _tc_forward(
        emb, pos_t, Wqkv, Wo, router,
        W1.astype(jnp.bfloat16), W2.astype(jnp.bfloat16),
        ln_g[0:1], ln_b[0:1],
        ln_g[1:2 * L:2].reshape(L, 1, D), ln_b[1:2 * L:2].reshape(L, 1, D),
        ln_g[2:2 * L + 1:2].reshape(L, 1, D),
        ln_b[2:2 * L + 1:2].reshape(L, 1, D),
        ln_g[2 * L + 1:], ln_b[2 * L + 1:])
    return out.reshape(B, T, D)


# speedup vs baseline: 1.2100x; 1.2100x over previous
import functools

import jax
import jax.numpy as jnp
from jax import lax
from jax.experimental import pallas as pl
from jax.experimental.pallas import tpu as pltpu
from jax.experimental.pallas import tpu_sc as plsc

B, T, D, H, L, E, K, V, DFF = 8, 128, 128, 4, 4, 16, 8, 262144, 512
HD = D // H
N = B * T

_NC, _NS = 2, 16
_NW = _NC * _NS
_BPW = N // _NW


def _sc_gather(table, idx):
    mesh = plsc.VectorSubcoreMesh(
        core_axis_name="c", subcore_axis_name="s",
        num_cores=_NC, num_subcores=_NS)

    @functools.partial(
        pl.kernel,
        out_type=jax.ShapeDtypeStruct((N, D), jnp.float32),
        mesh=mesh,
        scratch_types=[
            pltpu.VMEM((_BPW,), jnp.int32),
            pltpu.VMEM((_BPW, D), jnp.float32),
            pltpu.SemaphoreType.DMA,
        ],
    )
    def gather_kernel(table_hbm, idx_hbm, out_hbm, idx_v, rows_v, sem):
        wid = lax.axis_index("s") * _NC + lax.axis_index("c")
        base = wid * _BPW
        pltpu.sync_copy(idx_hbm.at[pl.ds(base, _BPW)], idx_v)
        pltpu.async_copy(table_hbm.at[idx_v], rows_v, sem).wait()
        pltpu.sync_copy(rows_v, out_hbm.at[pl.ds(base, _BPW)])

    return gather_kernel(table, idx)


def _ln(x, g, b):
    m = jnp.mean(x, axis=-1, keepdims=True)
    c = x - m
    v = jnp.mean(c * c, axis=-1, keepdims=True)
    return c * lax.rsqrt(v + 1e-5) * g + b


def _tc_body(emb_ref, pos_ref, wqkv_ref, wo_ref, rtr_ref, w1_ref, w2_ref,
             ln0g_ref, ln0b_ref, lnag_ref, lnab_ref, lnbg_ref, lnbb_ref,
             lnfg_ref, lnfb_ref, out_ref, h_ref):
    l = pl.program_id(0)

    @pl.when(l == 0)
    def _():
        h_ref[...] = _ln(emb_ref[...] + pos_ref[...],
                         ln0g_ref[...], ln0b_ref[...])

    h = h_ref[...]
    x = _ln(h, lnag_ref[0], lnab_ref[0])
    wqkv = wqkv_ref[0]
    wo = wo_ref[0]
    row = lax.broadcasted_iota(jnp.int32, (T, T), 0)
    col = lax.broadcasted_iota(jnp.int32, (T, T), 1)
    causal = row >= col
    scale = 1.0 / (HD ** 0.5)
    for b in range(B):
        xb = x[b * T:(b + 1) * T]
        qkv = jnp.dot(xb, wqkv, preferred_element_type=jnp.float32)
        obs = []
        for hh in range(H):
            q = qkv[:, hh * HD:(hh + 1) * HD]
            k = qkv[:, D + hh * HD:D + (hh + 1) * HD]
            v = qkv[:, 2 * D + hh * HD:2 * D + (hh + 1) * HD]
            att = lax.dot_general(q, k, (((1,), (1,)), ((), ())),
                                  preferred_element_type=jnp.float32) * scale
            att = jnp.where(causal, att, jnp.float32(-1e9))
            att = jax.nn.softmax(att, axis=-1)
            obs.append(jnp.dot(att, v, preferred_element_type=jnp.float32))
        ob = jnp.concatenate(obs, axis=-1)
        h_ref[b * T:(b + 1) * T, :] = (
            h[b * T:(b + 1) * T]
            + jnp.dot(ob, wo, preferred_element_type=jnp.float32))

    h = h_ref[...]
    x = _ln(h, lnbg_ref[0], lnbb_ref[0])
    logits = jnp.dot(x, rtr_ref[0], preferred_element_type=jnp.float32)
    probs = jax.nn.softmax(logits, axis=-1)
    masked = probs
    sel = jnp.zeros_like(probs)
    pos16 = lax.broadcasted_iota(jnp.int32, (N, E), 1)
    for _ in range(K):
        m = jnp.max(masked, axis=-1, keepdims=True)
        cand = masked == m
        candpos = jnp.where(cand, pos16, E)
        first = candpos == jnp.min(candpos, axis=-1, keepdims=True)
        sel = jnp.where(first, probs, sel)
        masked = jnp.where(first, jnp.float32(-1.0), masked)
    gates = sel / (jnp.sum(sel, axis=-1, keepdims=True) + 1e-9)

    acc = h
    for e in range(E):
        mid = jax.nn.gelu(jnp.dot(x, w1_ref[0, e],
                                  preferred_element_type=jnp.float32))
        eo = jnp.dot(mid, w2_ref[0, e], preferred_element_type=jnp.float32)
        acc = acc + eo * gates[:, e:e + 1]
    h_ref[...] = acc

    @pl.when(l == L - 1)
    def _():
        out_ref[...] = _ln(acc, lnfg_ref[...], lnfb_ref[...])


def _tc_forward(emb, pos_t, Wqkv, Wo, router, W1, W2,
                ln0g, ln0b, lnag, lnab, lnbg, lnbb, lnfg, lnfb,
                interpret=False):
    full2d = pl.BlockSpec((N, D), lambda l: (0, 0))
    perl = lambda shape: pl.BlockSpec(shape, lambda l: (l,) + (0,) * (len(shape) - 1))
    row0 = pl.BlockSpec((1, D), lambda l: (0, 0))
    rowl = pl.BlockSpec((1, 1, D), lambda l: (l, 0, 0))
    return pl.pallas_call(
        _tc_body,
        grid=(L,),
        in_specs=[
            full2d, full2d,
            perl((1, D, 3 * D)),
            perl((1, D, D)),
            perl((1, D, E)),
            perl((1, E, D, DFF)),
            perl((1, E, DFF, D)),
            row0, row0, rowl, rowl, rowl, rowl, row0, row0,
        ],
        out_specs=full2d,
        out_shape=jax.ShapeDtypeStruct((N, D), jnp.float32),
        scratch_shapes=[pltpu.VMEM((N, D), jnp.float32)],
        interpret=interpret,
    )(emb, pos_t, Wqkv, Wo, router, W1, W2,
      ln0g, ln0b, lnag, lnab, lnbg, lnbb, lnfg, lnfb)


def kernel(input_ids, tok_emb, pos_emb, Wqkv, Wo, router, W1, W2, ln_g, ln_b):
    ids = input_ids.reshape(N).astype(jnp.int32)
    emb = _sc_gather(tok_emb, ids)
    pos_t = jnp.broadcast_to(pos_emb[None], (B, T, D)).reshape(N, D)
    out = _tc_forward(
        emb, pos_t, Wqkv, Wo, router, W1, W2,
        ln_g[0:1], ln_b[0:1],
        ln_g[1:2 * L:2].reshape(L, 1, D), ln_b[1:2 * L:2].reshape(L, 1, D),
        ln_g[2:2 * L + 1:2].reshape(L, 1, D),
        ln_b[2:2 * L + 1:2].reshape(L, 1, D),
        ln_g[2 * L + 1:], ln_b[2 * L + 1:])
    return out.reshape(B, T, D)


# speedup vs baseline: 1.6904x; 1.3970x over previous
import functools

import jax
import jax.numpy as jnp
from jax import lax
from jax.experimental import pallas as pl
from jax.experimental.pallas import tpu as pltpu
from jax.experimental.pallas import tpu_sc as plsc

B, T, D, H, L, E, K, V, DFF = 8, 128, 128, 4, 4, 16, 8, 262144, 512
HD = D // H
N = B * T

_NC, _NS = 2, 16
_NW = _NC * _NS
_BPW = N // _NW


def _sc_gather(table, idx):
    mesh = plsc.VectorSubcoreMesh(
        core_axis_name="c", subcore_axis_name="s",
        num_cores=_NC, num_subcores=_NS)

    @functools.partial(
        pl.kernel,
        out_type=jax.ShapeDtypeStruct((N, D), jnp.float32),
        mesh=mesh,
        scratch_types=[
            pltpu.VMEM((_BPW,), jnp.int32),
            pltpu.VMEM((_BPW, D), jnp.float32),
            pltpu.SemaphoreType.DMA,
        ],
    )
    def gather_kernel(table_hbm, idx_hbm, out_hbm, idx_v, rows_v, sem):
        wid = lax.axis_index("s") * _NC + lax.axis_index("c")
        base = wid * _BPW
        pltpu.sync_copy(idx_hbm.at[pl.ds(base, _BPW)], idx_v)
        pltpu.async_copy(table_hbm.at[idx_v], rows_v, sem).wait()
        pltpu.sync_copy(rows_v, out_hbm.at[pl.ds(base, _BPW)])

    return gather_kernel(table, idx)


def _ln(x, g, b):
    m = jnp.mean(x, axis=-1, keepdims=True)
    c = x - m
    v = jnp.mean(c * c, axis=-1, keepdims=True)
    return c * lax.rsqrt(v + 1e-5) * g + b


def _tc_body(emb_ref, pos_ref, band_ref, cmask_ref,
             wqkv_ref, wo_ref, rtr_ref, w1_ref, w2_ref,
             ln0g_ref, ln0b_ref, lnag_ref, lnab_ref, lnbg_ref, lnbb_ref,
             lnfg_ref, lnfb_ref, out_ref, h_ref):
    l = pl.program_id(0)

    @pl.when(l == 0)
    def _():
        h_ref[...] = _ln(emb_ref[...] + pos_ref[...],
                         ln0g_ref[...], ln0b_ref[...])

    h = h_ref[...]
    x = _ln(h, lnag_ref[0], lnab_ref[0])
    qkv = jnp.dot(x, wqkv_ref[0], preferred_element_type=jnp.float32)
    band = band_ref[...]
    cmask = cmask_ref[...]
    obs = []
    for b in range(B):
        qb = qkv[b * T:(b + 1) * T, 0:D]
        kb = qkv[b * T:(b + 1) * T, D:2 * D]
        vb = qkv[b * T:(b + 1) * T, 2 * D:3 * D]
        qbd = jnp.concatenate([qb] * H, axis=0) * band
        att = lax.dot_general(qbd, kb, (((1,), (1,)), ((), ())),
                              preferred_element_type=jnp.float32) + cmask
        p = jax.nn.softmax(att, axis=-1)
        ofull = jnp.dot(p, vb, preferred_element_type=jnp.float32)
        obs.append(jnp.concatenate(
            [ofull[hh * T:(hh + 1) * T, hh * HD:(hh + 1) * HD]
             for hh in range(H)], axis=1))
    ob = jnp.concatenate(obs, axis=0)
    h = h + jnp.dot(ob, wo_ref[0], preferred_element_type=jnp.float32)
    h_ref[...] = h

    x = _ln(h, lnbg_ref[0], lnbb_ref[0])
    logits = jnp.dot(x, rtr_ref[0], preferred_element_type=jnp.float32)
    probs = jax.nn.softmax(logits, axis=-1)
    masked = probs
    sel = jnp.zeros_like(probs)
    pos16 = lax.broadcasted_iota(jnp.int32, (N, E), 1)
    for _ in range(K):
        m = jnp.max(masked, axis=-1, keepdims=True)
        cand = masked == m
        candpos = jnp.where(cand, pos16, E)
        first = candpos == jnp.min(candpos, axis=-1, keepdims=True)
        sel = jnp.where(first, probs, sel)
        masked = jnp.where(first, jnp.float32(-1.0), masked)
    gates = sel / (jnp.sum(sel, axis=-1, keepdims=True) + 1e-9)

    acc = h
    for e in range(E):
        mid = jax.nn.gelu(jnp.dot(x, w1_ref[0, e],
                                  preferred_element_type=jnp.float32))
        eo = jnp.dot(mid, w2_ref[0, e], preferred_element_type=jnp.float32)
        acc = acc + eo * gates[:, e:e + 1]
    h_ref[...] = acc

    @pl.when(l == L - 1)
    def _():
        out_ref[...] = _ln(acc, lnfg_ref[...], lnfb_ref[...])


def _tc_forward(emb, pos_t, band, cmask, Wqkv, Wo, router, W1, W2,
                ln0g, ln0b, lnag, lnab, lnbg, lnbb, lnfg, lnfb,
                interpret=False):
    full2d = pl.BlockSpec((N, D), lambda l: (0, 0))
    stk = pl.BlockSpec((H * T, T), lambda l: (0, 0))
    perl = lambda shape: pl.BlockSpec(shape, lambda l: (l,) + (0,) * (len(shape) - 1))
    row0 = pl.BlockSpec((1, D), lambda l: (0, 0))
    rowl = pl.BlockSpec((1, 1, D), lambda l: (l, 0, 0))
    return pl.pallas_call(
        _tc_body,
        grid=(L,),
        in_specs=[
            full2d, full2d, stk, stk,
            perl((1, D, 3 * D)),
            perl((1, D, D)),
            perl((1, D, E)),
            perl((1, E, D, DFF)),
            perl((1, E, DFF, D)),
            row0, row0, rowl, rowl, rowl, rowl, row0, row0,
        ],
        out_specs=full2d,
        out_shape=jax.ShapeDtypeStruct((N, D), jnp.float32),
        scratch_shapes=[pltpu.VMEM((N, D), jnp.float32)],
        interpret=interpret,
    )(emb, pos_t, band, cmask, Wqkv, Wo, router, W1, W2,
      ln0g, ln0b, lnag, lnab, lnbg, lnbb, lnfg, lnfb)


def _masks():
    import numpy as np
    r = np.arange(H * T)
    band = ((r[:, None] // T) == (np.arange(D)[None, :] // HD))
    band = band.astype(np.float32) / (HD ** 0.5)
    cmask = np.where((r[:, None] % T) >= np.arange(T)[None, :],
                     0.0, -1e9).astype(np.float32)
    return jnp.asarray(band), jnp.asarray(cmask)


def kernel(input_ids, tok_emb, pos_emb, Wqkv, Wo, router, W1, W2, ln_g, ln_b):
    ids = input_ids.reshape(N).astype(jnp.int32)
    emb = _sc_gather(tok_emb, ids)
    pos_t = jnp.broadcast_to(pos_emb[None], (B, T, D)).reshape(N, D)
    band, cmask = _masks()
    out = _tc_forward(
        emb, pos_t, band, cmask, Wqkv, Wo, router, W1, W2,
        ln_g[0:1], ln_b[0:1],
        ln_g[1:2 * L:2].reshape(L, 1, D), ln_b[1:2 * L:2].reshape(L, 1, D),
        ln_g[2:2 * L + 1:2].reshape(L, 1, D),
        ln_b[2:2 * L + 1:2].reshape(L, 1, D),
        ln_g[2 * L + 1:], ln_b[2 * L + 1:])
    return out.reshape(B, T, D)
